# baseline (speedup 1.0000x reference)
import numpy as np
import jax
import jax.numpy as jnp
from jax import lax
from jax.experimental import pallas as pl
from jax.experimental.pallas import tpu as pltpu
from jax.experimental.pallas import tpu_sc as plsc

D_MODEL = 64
MAX_SEQ_LEN = 200
BATCH = 4096

_G = 8
_NGRP = MAX_SEQ_LEN // _G
_GW = _G * D_MODEL
_ZROW = MAX_SEQ_LEN

_NC = 2
_NS = 16
_NW = _NC * _NS
_ROWS_PER_W = BATCH // _NW
_ENT_PER_W = _ROWS_PER_W * _NGRP
_CHUNK = 64
_NCHUNKS = _ENT_PER_W // _CHUNK
_POS_PER_CHUNK = _CHUNK * _G
_L = 16

_MASK = np.zeros((_G, _GW), dtype=np.float32)
for _c in range(1, _G + 1):
    _MASK[_c - 1, :_c * D_MODEL] = 1.0


def _pe_body(len_hbm, gtab_hbm, emb_hbm, pos_hbm,
             len_v, gidx_v, pid_v, rows0, rows1,
             sem_g0, sem_g1, sem_w0, sem_w1, sem_p):
    wid = lax.axis_index("s") * _NC + lax.axis_index("c")
    ent_base = wid * _ENT_PER_W
    pos_base = wid * _ROWS_PER_W * MAX_SEQ_LEN
    pltpu.sync_copy(len_hbm.at[pl.ds(wid * _ROWS_PER_W, _ROWS_PER_W)], len_v)

    def chunk(j, carry):
        e0 = j * _CHUNK
        for v in range(_CHUNK // _L):
            e = e0 + v * _L + lax.iota(jnp.int32, _L)
            r = e // _NGRP
            t = e - r * _NGRP
            lenr = plsc.load_gather(len_v, [r])
            kept = jnp.clip(lenr - _G * t, 0, _G)
            gidx_v[j, pl.ds(v * _L, _L)] = jnp.where(
                kept >= 1, _G * t + kept - 1, _ZROW)
        p0 = e0 * _G
        for v in range(_POS_PER_CHUNK // _L):
            p = p0 + v * _L + lax.iota(jnp.int32, _L)
            r = p // MAX_SEQ_LEN
            pos = p - r * MAX_SEQ_LEN + 1
            lenr = plsc.load_gather(len_v, [r])
            pid_v[j, pl.ds(v * _L, _L)] = jnp.where(pos <= lenr, pos, 0)
        return carry

    lax.fori_loop(0, _NCHUNKS, chunk, 0)

    rows = (rows0, rows1)
    sem_g = (sem_g0, sem_g1)
    sem_w = (sem_w0, sem_w1)

    def g_start(c):
        return pltpu.async_copy(gtab_hbm.at[gidx_v.at[c]], rows[c % 2],
                                sem_g[c % 2])

    g_handles = {0: g_start(0), 1: g_start(1)}
    w_handles = {}
    p_handles = {}
    for c in range(_NCHUNKS):
        b = c % 2
        g_handles.pop(c).wait()
        w_handles[c] = pltpu.async_copy(
            rows[b], emb_hbm.at[pl.ds(ent_base + c * _CHUNK, _CHUNK)],
            sem_w[b])
        p_handles[c] = pltpu.async_copy(
            pid_v.at[c],
            pos_hbm.at[pl.ds(pos_base + c * _POS_PER_CHUNK, _POS_PER_CHUNK)],
            sem_p)
        if c >= 2:
            p_handles.pop(c - 2).wait()
        if c + 2 < _NCHUNKS:
            w_handles.pop(c).wait()
            g_handles[c + 2] = g_start(c + 2)
    for h in (*w_handles.values(), *p_handles.values()):
        h.wait()


def kernel(input_len, device, table):
    del device
    grouped = jnp.concatenate(
        [(table[1:].reshape(_NGRP, 1, _GW) * _MASK[None]).reshape(
            MAX_SEQ_LEN, _GW),
         jnp.zeros((1, _GW), jnp.float32)], axis=0)
    mesh = plsc.VectorSubcoreMesh(core_axis_name="c", subcore_axis_name="s")
    k = pl.kernel(
        _pe_body,
        mesh=mesh,
        compiler_params=pltpu.CompilerParams(needs_layout_passes=False),
        out_type=[
            jax.ShapeDtypeStruct((BATCH * _NGRP, _GW), jnp.float32),
            jax.ShapeDtypeStruct((BATCH * MAX_SEQ_LEN,), jnp.int32),
        ],
        scratch_types=[
            pltpu.VMEM((_ROWS_PER_W,), jnp.int32),
            pltpu.VMEM((_NCHUNKS, _CHUNK), jnp.int32),
            pltpu.VMEM((_NCHUNKS, _POS_PER_CHUNK), jnp.int32),
            pltpu.VMEM((_CHUNK, _GW), jnp.float32),
            pltpu.VMEM((_CHUNK, _GW), jnp.float32),
            pltpu.SemaphoreType.DMA,
            pltpu.SemaphoreType.DMA,
            pltpu.SemaphoreType.DMA,
            pltpu.SemaphoreType.DMA,
            pltpu.SemaphoreType.DMA,
        ],
    )
    emb_flat, pos_flat = k(input_len.astype(jnp.int32), grouped)
    return (emb_flat.reshape(BATCH, MAX_SEQ_LEN, D_MODEL),
            pos_flat.reshape(BATCH, MAX_SEQ_LEN))


# speedup vs baseline: 4.7997x; 4.7997x over previous
import jax
import jax.numpy as jnp
from jax import lax
from jax.experimental import pallas as pl
from jax.experimental.pallas import tpu as pltpu
from jax.experimental.pallas import tpu_sc as plsc

D_MODEL = 64
MAX_SEQ_LEN = 200
BATCH = 4096

_NC = 2
_NS = 16
_NW = _NC * _NS
_ROWS_PER_W = BATCH // _NW
_POS_PER_W = _ROWS_PER_W * MAX_SEQ_LEN
_L = 16
_ZMAX = 128
_BITS = (128, 64, 32, 16, 8, 4, 2, 1)
_TABW = MAX_SEQ_LEN * D_MODEL
_ROWW = MAX_SEQ_LEN * D_MODEL


def _pe_body(len_hbm, table_hbm, emb_hbm, pos_hbm,
             len_v, tab_v, zero_v, pid_v, sem_e, sem_p):
    wid = lax.axis_index("s") * _NC + lax.axis_index("c")
    row_base = wid * _ROWS_PER_W
    pos_base = wid * _POS_PER_W
    pltpu.sync_copy(len_hbm.at[pl.ds(row_base, _ROWS_PER_W)],
                    len_v.at[pl.ds(0, _ROWS_PER_W)])
    pltpu.sync_copy(table_hbm.at[pl.ds(D_MODEL, _TABW)], tab_v)

    zvec = jnp.zeros((_L,), jnp.float32)

    def zfill(i, carry):
        zero_v[pl.ds(i * _L, _L)] = zvec
        return carry

    lax.fori_loop(0, _ZMAX * D_MODEL // _L, zfill, 0)

    def row(r, carry):
        ln = len_v[pl.ds(r, _L)][0]
        out0 = (row_base + r) * _ROWW
        off = jnp.int32(0)
        for bit in _BITS:
            @pl.when((ln & bit) != 0)
            def _(off=off, bit=bit):
                pltpu.async_copy(
                    tab_v.at[pl.ds(off * D_MODEL, bit * D_MODEL)],
                    emb_hbm.at[pl.ds(out0 + off * D_MODEL, bit * D_MODEL)],
                    sem_e)
            off = off + (ln & bit)
        rem = MAX_SEQ_LEN - ln
        for bit in _BITS:
            @pl.when((rem & bit) != 0)
            def _(off=off, bit=bit):
                pltpu.async_copy(
                    zero_v.at[pl.ds(0, bit * D_MODEL)],
                    emb_hbm.at[pl.ds(out0 + off * D_MODEL, bit * D_MODEL)],
                    sem_e)
            off = off + (rem & bit)
        return carry

    lax.fori_loop(0, _ROWS_PER_W, row, 0)

    def pchunk(v, carry):
        p = v * _L + lax.iota(jnp.int32, _L)
        r = p // MAX_SEQ_LEN
        pos = p - r * MAX_SEQ_LEN + 1
        lenr = plsc.load_gather(len_v, [r])
        pid_v[pl.ds(v * _L, _L)] = jnp.where(pos <= lenr, pos, 0)
        return carry

    lax.fori_loop(0, _POS_PER_W // _L, pchunk, 0)
    pltpu.async_copy(pid_v, pos_hbm.at[pl.ds(pos_base, _POS_PER_W)],
                     sem_p).wait()

    def drain(r, carry):
        pltpu.make_async_copy(
            emb_hbm.at[pl.ds((row_base + r) * _ROWW, _ROWW)],
            tab_v, sem_e).wait()
        return carry

    lax.fori_loop(0, _ROWS_PER_W, drain, 0)


def kernel(input_len, device, table):
    del device
    mesh = plsc.VectorSubcoreMesh(core_axis_name="c", subcore_axis_name="s")
    k = pl.kernel(
        _pe_body,
        mesh=mesh,
        compiler_params=pltpu.CompilerParams(needs_layout_passes=False),
        out_type=[
            jax.ShapeDtypeStruct((BATCH * MAX_SEQ_LEN * D_MODEL,),
                                 jnp.float32),
            jax.ShapeDtypeStruct((BATCH * MAX_SEQ_LEN,), jnp.int32),
        ],
        scratch_types=[
            pltpu.VMEM((_ROWS_PER_W + _L,), jnp.int32),
            pltpu.VMEM((_TABW,), jnp.float32),
            pltpu.VMEM((_ZMAX * D_MODEL,), jnp.float32),
            pltpu.VMEM((_POS_PER_W,), jnp.int32),
            pltpu.SemaphoreType.DMA,
            pltpu.SemaphoreType.DMA,
        ],
    )
    emb_flat, pos_flat = k(input_len.astype(jnp.int32),
                           table.reshape(-1))
    return (emb_flat.reshape(BATCH, MAX_SEQ_LEN, D_MODEL),
            pos_flat.reshape(BATCH, MAX_SEQ_LEN))
